# baseline (speedup 1.0000x reference)
import functools

import jax
import jax.numpy as jnp
from jax.experimental import pallas as pl
from jax.experimental.pallas import tpu as pltpu

_TN = 512


def _chamfer_tile_kernel(a_ref, b_ref, xx_ref, yy_ref, xmin_ref, ymin8_ref):
    i = pl.program_id(1)
    TN = a_ref.shape[1]
    M = b_ref.shape[2]
    xxc = xx_ref[0]
    g = jnp.dot(a_ref[0], b_ref[0], preferred_element_type=jnp.float32)
    e = g + yy_ref[0]
    xs = [e[:, k * 128:(k + 1) * 128] for k in range(M // 128)]
    while len(xs) > 1:
        xs = [
            jnp.minimum(xs[2 * t], xs[2 * t + 1]) if 2 * t + 1 < len(xs) else xs[2 * t]
            for t in range((len(xs) + 1) // 2)
        ]
    xmin_ref[0, 0, 0, :] = jnp.min(xs[0], axis=1)
    f = g + xxc
    ys = [f[r * 8:(r + 1) * 8, :] for r in range(TN // 8)]
    while len(ys) > 1:
        ys = [
            jnp.minimum(ys[2 * t], ys[2 * t + 1]) if 2 * t + 1 < len(ys) else ys[2 * t]
            for t in range((len(ys) + 1) // 2)
        ]
    yj = ys[0][None]

    @pl.when(i == 0)
    def _init():
        ymin8_ref[...] = yj

    @pl.when(i > 0)
    def _acc():
        ymin8_ref[...] = jnp.minimum(ymin8_ref[...], yj)


@functools.partial(jax.jit, static_argnames=())
def _chamfer(predict, target):
    B, N, _ = predict.shape
    _, M, _ = target.shape
    f32 = jnp.float32
    bf16 = jnp.bfloat16

    xx = jnp.sum(predict * predict, axis=-1, keepdims=True)
    ty = target.transpose(0, 2, 1)
    yy = jnp.sum(ty * ty, axis=1, keepdims=True)
    amat = predict.astype(bf16)
    bmat = (-2.0 * ty).astype(bf16)

    nb = N // _TN
    x_part, y_part8 = pl.pallas_call(
        _chamfer_tile_kernel,
        grid=(B, nb),
        in_specs=[
            pl.BlockSpec((1, _TN, 3), lambda b, i: (b, i, 0)),
            pl.BlockSpec((1, 3, M), lambda b, i: (b, 0, 0)),
            pl.BlockSpec((1, _TN, 1), lambda b, i: (b, i, 0)),
            pl.BlockSpec((1, 1, M), lambda b, i: (b, 0, 0)),
        ],
        out_specs=[
            pl.BlockSpec((1, 1, 1, _TN), lambda b, i: (b, i, 0, 0)),
            pl.BlockSpec((1, 8, M), lambda b, i: (b, 0, 0)),
        ],
        out_shape=[
            jax.ShapeDtypeStruct((B, nb, 1, _TN), f32),
            jax.ShapeDtypeStruct((B, 8, M), f32),
        ],
        compiler_params=pltpu.CompilerParams(
            dimension_semantics=("parallel", "arbitrary"),
        ),
    )(amat, bmat, xx, yy)
    x_near = x_part.reshape(B, N) + xx[:, :, 0]
    y_near = jnp.min(y_part8, axis=1) + yy[:, 0, :]
    return x_near.mean() + y_near.mean()


def kernel(predict, target):
    return _chamfer(predict, target)


# speedup vs baseline: 1.1354x; 1.1354x over previous
import functools

import jax
import jax.numpy as jnp
from jax.experimental import pallas as pl
from jax.experimental.pallas import tpu as pltpu

_TN = 512


def _chamfer_tile_kernel(a_ref, b_ref, xx_ref, yy_ref, xmin_ref, ymin8_ref):
    i = pl.program_id(1)
    TN = a_ref.shape[1]
    M = b_ref.shape[2]
    xxc = xx_ref[0]
    g = jnp.dot(a_ref[0], b_ref[0], preferred_element_type=jnp.float32)
    e = g + yy_ref[0]
    xs = [e[:, k * 128:(k + 1) * 128] for k in range(M // 128)]
    while len(xs) > 1:
        xs = [
            jnp.minimum(xs[2 * t], xs[2 * t + 1]) if 2 * t + 1 < len(xs) else xs[2 * t]
            for t in range((len(xs) + 1) // 2)
        ]
    xmin_ref[0, 0] = xs[0]
    f = g + xxc
    ys = [f[r * 8:(r + 1) * 8, :] for r in range(TN // 8)]
    while len(ys) > 1:
        ys = [
            jnp.minimum(ys[2 * t], ys[2 * t + 1]) if 2 * t + 1 < len(ys) else ys[2 * t]
            for t in range((len(ys) + 1) // 2)
        ]
    yj = ys[0][None]

    @pl.when(i == 0)
    def _init():
        ymin8_ref[...] = yj

    @pl.when(i > 0)
    def _acc():
        ymin8_ref[...] = jnp.minimum(ymin8_ref[...], yj)


def _finalize_kernel(xpart_ref, ypart_ref, xx_ref, yy_ref, out_ref):
    B, nb, TN, _ = xpart_ref.shape
    M = ypart_ref.shape[2]
    N = nb * TN
    xp = xpart_ref[...]
    x_near = jnp.min(xp, axis=-1)[..., None] + xx_ref[...].reshape(B, nb, TN, 1)
    yp = ypart_ref[...]
    y8 = jnp.minimum(
        jnp.minimum(jnp.minimum(yp[:, 0:2], yp[:, 2:4]), yp[:, 4:6]), yp[:, 6:8]
    )
    y_near = jnp.minimum(y8[:, 0:1], y8[:, 1:2]) + yy_ref[...]
    total = jnp.sum(x_near) / N + jnp.sum(y_near) / M
    out_ref[...] = total.reshape(1, 1)


@functools.partial(jax.jit, static_argnames=())
def _chamfer(predict, target):
    B, N, _ = predict.shape
    _, M, _ = target.shape
    f32 = jnp.float32
    bf16 = jnp.bfloat16

    xx = jnp.sum(predict * predict, axis=-1, keepdims=True)
    ty = target.transpose(0, 2, 1)
    yy = jnp.sum(ty * ty, axis=1, keepdims=True)
    amat = predict.astype(bf16)
    bmat = (-2.0 * ty).astype(bf16)

    nb = N // _TN
    x_part, y_part8 = pl.pallas_call(
        _chamfer_tile_kernel,
        grid=(B, nb),
        in_specs=[
            pl.BlockSpec((1, _TN, 3), lambda b, i: (b, i, 0)),
            pl.BlockSpec((1, 3, M), lambda b, i: (b, 0, 0)),
            pl.BlockSpec((1, _TN, 1), lambda b, i: (b, i, 0)),
            pl.BlockSpec((1, 1, M), lambda b, i: (b, 0, 0)),
        ],
        out_specs=[
            pl.BlockSpec((1, 1, _TN, 128), lambda b, i: (b, i, 0, 0)),
            pl.BlockSpec((1, 8, M), lambda b, i: (b, 0, 0)),
        ],
        out_shape=[
            jax.ShapeDtypeStruct((B, nb, _TN, 128), f32),
            jax.ShapeDtypeStruct((B, 8, M), f32),
        ],
        compiler_params=pltpu.CompilerParams(
            dimension_semantics=("parallel", "arbitrary"),
        ),
    )(amat, bmat, xx, yy)

    total = pl.pallas_call(
        _finalize_kernel,
        out_shape=jax.ShapeDtypeStruct((1, 1), f32),
    )(x_part, y_part8, xx, yy)
    return total[0, 0] / B


def kernel(predict, target):
    return _chamfer(predict, target)


# speedup vs baseline: 1.1905x; 1.0485x over previous
import functools

import jax
import jax.numpy as jnp
from jax.experimental import pallas as pl
from jax.experimental.pallas import tpu as pltpu

_TN = 512


def _chamfer_tile_kernel(a_ref, b_ref, xx_ref, yy_ref, xmin_ref, ymin8_ref):
    i = pl.program_id(1)
    TN = a_ref.shape[1]
    M = b_ref.shape[2]
    a = a_ref[0]
    xxc = xx_ref[0]
    g = jnp.dot(a, b_ref[0], preferred_element_type=jnp.float32)
    e = g + yy_ref[0]
    xs = [e[:, k * 128:(k + 1) * 128] for k in range(M // 128)]
    while len(xs) > 1:
        xs = [
            jnp.minimum(xs[2 * t], xs[2 * t + 1]) if 2 * t + 1 < len(xs) else xs[2 * t]
            for t in range((len(xs) + 1) // 2)
        ]
    xacc = xs[0]
    f = g + xxc
    ys = [f[r * 8:(r + 1) * 8, :] for r in range(TN // 8)]
    while len(ys) > 1:
        ys = [
            jnp.minimum(ys[2 * t], ys[2 * t + 1]) if 2 * t + 1 < len(ys) else ys[2 * t]
            for t in range((len(ys) + 1) // 2)
        ]
    yj = ys[0][None]

    @pl.when(i == 0)
    def _init():
        ymin8_ref[...] = yj

    @pl.when(i > 0)
    def _acc():
        ymin8_ref[...] = jnp.minimum(ymin8_ref[...], yj)

    xmin_ref[0, 0] = xacc


@functools.partial(jax.jit, static_argnames=())
def _chamfer(predict, target):
    B, N, _ = predict.shape
    _, M, _ = target.shape
    f32 = jnp.float32
    bf16 = jnp.bfloat16

    xx = jnp.sum(predict * predict, axis=-1, keepdims=True)
    amat = predict.astype(bf16)
    ty = target.transpose(0, 2, 1)
    yy = jnp.sum(ty * ty, axis=1, keepdims=True)
    bmat = (-2.0 * ty).astype(bf16)

    nb = N // _TN
    x_part, y_part8 = pl.pallas_call(
        _chamfer_tile_kernel,
        grid=(B, nb),
        in_specs=[
            pl.BlockSpec((1, _TN, 3), lambda b, i: (b, i, 0)),
            pl.BlockSpec((1, 3, M), lambda b, i: (b, 0, 0)),
            pl.BlockSpec((1, _TN, 1), lambda b, i: (b, i, 0)),
            pl.BlockSpec((1, 1, M), lambda b, i: (b, 0, 0)),
        ],
        out_specs=[
            pl.BlockSpec((1, 1, _TN, 128), lambda b, i: (b, i, 0, 0)),
            pl.BlockSpec((1, 8, M), lambda b, i: (b, 0, 0)),
        ],
        out_shape=[
            jax.ShapeDtypeStruct((B, nb, _TN, 128), f32),
            jax.ShapeDtypeStruct((B, 8, M), f32),
        ],
        compiler_params=pltpu.CompilerParams(
            dimension_semantics=("parallel", "arbitrary"),
        ),
    )(amat, bmat, xx, yy)
    x_near = jnp.min(x_part, axis=-1).reshape(B, N) + xx[:, :, 0]
    y_near = jnp.min(y_part8, axis=1) + yy[:, 0, :]
    return x_near.mean() + y_near.mean()


def kernel(predict, target):
    return _chamfer(predict, target)
